# baseline (speedup 1.0000x reference)
import functools

import jax
import jax.numpy as jnp
from jax import lax
from jax.experimental import pallas as pl
from jax.experimental.pallas import tpu as pltpu
from jax.experimental.pallas import tpu_sc as plsc

_N = 10000
_E = 160000
_D = 256
_H = 128
_K = 128
_NT = 16
_NC = 2
_CPT = 80
_CP0 = 40
_NWIN = 2
_CPW = _CPT // _NWIN
_EPAD = _K * _NT * _CPT
_NROW = 10240
_RPT = _NROW // _NT
_BN = 640
_DW = 16


def _sc_degree_body(dst_hbm, ones_hbm, zeros_hbm, out_hbm, idx_v, ones_v, acc):
    c = lax.axis_index("c")
    s = lax.axis_index("s")
    r0 = s * _RPT
    pltpu.sync_copy(zeros_hbm.at[pl.ds(r0, _RPT)], acc.at[pl.ds(r0, _RPT)])
    pltpu.sync_copy(ones_hbm, ones_v)
    pltpu.sync_copy(dst_hbm.at[c, s], idx_v)
    plsc.subcore_barrier()
    for j in range(_CP0):
        pltpu.sync_copy(ones_v, acc.at[idx_v.at[j]], add=True)
    plsc.subcore_barrier()
    pltpu.sync_copy(acc.at[pl.ds(r0, _RPT)], out_hbm.at[c, pl.ds(r0, _RPT)])


def _sc_aggregate_body(h_hbm, src_hbm, dst_hbm, out_hbm,
                       src_v, dst_v, buf, acc, gsem0, gsem1, ssem0, ssem1):
    c = lax.axis_index("c")
    s = lax.axis_index("s")
    r0 = s * _RPT
    pltpu.sync_copy(h_hbm.at[pl.ds(c * _NROW + r0, _RPT)],
                    acc.at[pl.ds(r0, _RPT)])
    plsc.subcore_barrier()

    gsems = (gsem0, gsem1)
    ssems = (ssem0, ssem1)

    def _gather(j, b):
        pltpu.async_copy(h_hbm.at[src_v.at[j]], buf.at[b], gsems[b])

    def _gwait(j, b):
        pltpu.make_async_copy(h_hbm.at[src_v.at[j]], buf.at[b], gsems[b]).wait()

    def _sstart(j, b):
        pltpu.async_copy(buf.at[b], acc.at[dst_v.at[j]], ssems[b], add=True)

    def _swait(j, b):
        pltpu.make_async_copy(buf.at[b], acc.at[dst_v.at[j]], ssems[b]).wait()

    for win in range(_NWIN):
        pltpu.sync_copy(src_hbm.at[c, s, pl.ds(win * _CPW, _CPW)], src_v)
        pltpu.sync_copy(dst_hbm.at[s, pl.ds(win * _CPW, _CPW)], dst_v)
        _gather(0, 0)
        _gather(1, 1)

        def _body(i, carry):
            j = i * 2
            _gwait(j, 0)
            _sstart(j, 0)
            _gwait(j + 1, 1)
            _sstart(j + 1, 1)
            @pl.when(j + 2 < _CPW)
            def _():
                _swait(j, 0)
                _gather(j + 2, 0)
            @pl.when(j + 3 < _CPW)
            def _():
                _swait(j + 1, 1)
                _gather(j + 3, 1)
            return carry

        lax.fori_loop(0, _CPW // 2, _body, 0)
        _swait(_CPW - 2, 0)
        _swait(_CPW - 1, 1)
    plsc.subcore_barrier()
    pltpu.sync_copy(acc.at[pl.ds(r0, _RPT)], out_hbm.at[c, pl.ds(r0, _RPT)])


@functools.cache
def _sc_kernels():
    mesh = plsc.VectorSubcoreMesh(core_axis_name="c", subcore_axis_name="s")
    degree = pl.kernel(
        _sc_degree_body,
        out_type=jax.ShapeDtypeStruct((_NC, _NROW, _H), jnp.float32),
        mesh=mesh,
        scratch_types=[
            pltpu.VMEM((_CP0, _K), jnp.int32),
            pltpu.VMEM((_K, _H), jnp.float32),
            pltpu.VMEM_SHARED((_NROW, _H), jnp.float32),
        ],
    )
    aggregate = pl.kernel(
        _sc_aggregate_body,
        out_type=jax.ShapeDtypeStruct((_NC, _NROW, _H), jnp.float32),
        mesh=mesh,
        scratch_types=[
            pltpu.VMEM((_CPW, _K), jnp.int32),
            pltpu.VMEM((_CPW, _K), jnp.int32),
            pltpu.VMEM((2, _K, _H), jnp.float32),
            pltpu.VMEM_SHARED((_NROW, _H), jnp.float32),
            pltpu.SemaphoreType.DMA,
            pltpu.SemaphoreType.DMA,
            pltpu.SemaphoreType.DMA,
            pltpu.SemaphoreType.DMA,
        ],
    )
    return degree, aggregate


def _dinv_rows(dg_ref):
    deg = dg_ref[0, :, 0] + dg_ref[1, :, 0] + 1.0
    return lax.rsqrt(deg)[:, None]


def _tc_l1_body(dg_ref, x_ref, w_ref, out_ref):
    dinv = _dinv_rows(dg_ref)
    h = jnp.dot(x_ref[...], w_ref[...], preferred_element_type=jnp.float32)
    hs = h * dinv
    out_ref[0] = hs[:, :_H]
    out_ref[1] = hs[:, _H:]


def _tc_l2_body(dg_ref, agg_ref, b_ref, w_ref, out_ref):
    dinv = _dinv_rows(dg_ref)
    z = jnp.concatenate([agg_ref[0], agg_ref[1]], axis=1) * dinv + b_ref[...]
    h = jnp.dot(z, w_ref[...], preferred_element_type=jnp.float32) * dinv
    out_ref[0] = h[:, :_H]
    out_ref[1] = h[:, _H:]


def _tc_out_body(dg_ref, agg_ref, b_ref, out_ref):
    dinv = _dinv_rows(dg_ref)
    out_ref[...] = (jnp.concatenate([agg_ref[0], agg_ref[1]], axis=1) * dinv
                    + b_ref[...])


_GRID = (_NROW // _BN,)
_dg_spec = pl.BlockSpec((2, _BN, _DW), lambda i: (0, i, 0))
_agg_spec = pl.BlockSpec((2, _BN, _H), lambda i: (0, i, 0))
_b_spec = pl.BlockSpec((1, _D), lambda i: (0, 0))
_w_spec = pl.BlockSpec((_D, _D), lambda i: (0, 0))

_tc_l1 = pl.pallas_call(
    _tc_l1_body,
    grid=_GRID,
    in_specs=[_dg_spec, pl.BlockSpec((_BN, _D), lambda i: (i, 0)), _w_spec],
    out_specs=_agg_spec,
    out_shape=jax.ShapeDtypeStruct((2, _NROW, _H), jnp.float32),
)

_tc_l2 = pl.pallas_call(
    _tc_l2_body,
    grid=_GRID,
    in_specs=[_dg_spec, _agg_spec, _b_spec, _w_spec],
    out_specs=_agg_spec,
    out_shape=jax.ShapeDtypeStruct((2, _NROW, _H), jnp.float32),
)

_tc_out = pl.pallas_call(
    _tc_out_body,
    grid=_GRID,
    in_specs=[_dg_spec, _agg_spec, _b_spec],
    out_specs=pl.BlockSpec((_BN, _D), lambda i: (i, 0)),
    out_shape=jax.ShapeDtypeStruct((_N, _D), jnp.float32),
)


def kernel(x, edge_index, W1, b1, W2, b2):
    src = edge_index[0].astype(jnp.int32)
    dst = edge_index[1].astype(jnp.int32)
    npad = _EPAD - _E
    src_p = jnp.concatenate([src, jnp.zeros((npad,), jnp.int32)])
    dst_p = jnp.concatenate([dst, jnp.full((npad,), _N, jnp.int32)])
    src_t = src_p.reshape(_NT, _CPT, _K)
    dst_t = dst_p.reshape(_NT, _CPT, _K)
    src2 = jnp.stack([src_t, src_t + _NROW])
    dst0 = dst_p.reshape(_NC, _NT, _CP0, _K)
    ones = jnp.ones((_K, _H), jnp.float32)
    zeros = jnp.zeros((_NROW, _H), jnp.float32)
    b1r = b1.reshape(1, _D)
    b2r = b2.reshape(1, _D)

    sc_degree, sc_aggregate = _sc_kernels()
    degf = sc_degree(dst0, ones, zeros)
    deg2 = lax.slice(degf, (0, 0, 0), (_NC, _NROW, _DW))
    h1 = _tc_l1(deg2, x, W1)
    agg1 = sc_aggregate(h1.reshape(_NC * _NROW, _H), src2, dst_t)
    h2 = _tc_l2(deg2, agg1, b1r, W2)
    agg2 = sc_aggregate(h2.reshape(_NC * _NROW, _H), src2, dst_t)
    return _tc_out(deg2, agg2, b2r)


# speedup vs baseline: 1.0627x; 1.0627x over previous
import functools

import jax
import jax.numpy as jnp
from jax import lax
from jax.experimental import pallas as pl
from jax.experimental.pallas import tpu as pltpu
from jax.experimental.pallas import tpu_sc as plsc

_N = 10000
_E = 160000
_D = 256
_H = 128
_K = 128
_NT = 16
_NC = 2
_CPT = 80
_CP0 = 40
_NWIN = 2
_CPW = _CPT // _NWIN
_EPAD = _K * _NT * _CPT
_NROW = 10240
_RPT = _NROW // _NT
_BN = 640
_DW = 16


def _sc_degree_body(dst_hbm, ones_hbm, zeros_hbm, out_hbm, idx_v, ones_v, acc, dsem):
    c = lax.axis_index("c")
    s = lax.axis_index("s")
    r0 = s * _RPT
    pltpu.sync_copy(zeros_hbm.at[pl.ds(r0, _RPT)], acc.at[pl.ds(r0, _RPT)])
    pltpu.sync_copy(ones_hbm, ones_v)
    pltpu.sync_copy(dst_hbm.at[c, s], idx_v)
    plsc.subcore_barrier()
    for j in range(_CP0):
        pltpu.async_copy(ones_v, acc.at[idx_v.at[j]], dsem, add=True)
    for j in range(_CP0):
        pltpu.make_async_copy(ones_v, acc.at[idx_v.at[j]], dsem).wait()
    plsc.subcore_barrier()
    pltpu.sync_copy(acc.at[pl.ds(r0, _RPT)], out_hbm.at[c, pl.ds(r0, _RPT)])


def _sc_aggregate_body(h_hbm, src_hbm, dst_hbm, out_hbm,
                       src_v, dst_v, buf, acc, sem0, sem1):
    c = lax.axis_index("c")
    s = lax.axis_index("s")
    r0 = s * _RPT
    pltpu.sync_copy(h_hbm.at[pl.ds(c * _NROW + r0, _RPT)],
                    acc.at[pl.ds(r0, _RPT)])
    plsc.subcore_barrier()

    sems = (sem0, sem1)

    def _gather(j, b):
        pltpu.async_copy(h_hbm.at[src_v.at[j]], buf.at[b], sems[b])

    def _gwait(j, b):
        pltpu.make_async_copy(h_hbm.at[src_v.at[j]], buf.at[b], sems[b]).wait()

    for win in range(_NWIN):
        pltpu.sync_copy(src_hbm.at[c, s, pl.ds(win * _CPW, _CPW)], src_v)
        pltpu.sync_copy(dst_hbm.at[s, pl.ds(win * _CPW, _CPW)], dst_v)
        _gather(0, 0)
        _gather(1, 1)
        for j in range(_CPW):
            b = j % 2
            _gwait(j, b)
            if j + 2 < _CPW:
                _gather(j + 2, b)
            pltpu.sync_copy(buf.at[b], acc.at[dst_v.at[j]], add=True)
    plsc.subcore_barrier()
    pltpu.sync_copy(acc.at[pl.ds(r0, _RPT)], out_hbm.at[c, pl.ds(r0, _RPT)])


@functools.cache
def _sc_kernels():
    mesh = plsc.VectorSubcoreMesh(core_axis_name="c", subcore_axis_name="s")
    degree = pl.kernel(
        _sc_degree_body,
        out_type=jax.ShapeDtypeStruct((_NC, _NROW, _H), jnp.float32),
        mesh=mesh,
        scratch_types=[
            pltpu.VMEM((_CP0, _K), jnp.int32),
            pltpu.VMEM((_K, _H), jnp.float32),
            pltpu.VMEM_SHARED((_NROW, _H), jnp.float32),
            pltpu.SemaphoreType.DMA,
        ],
    )
    aggregate = pl.kernel(
        _sc_aggregate_body,
        out_type=jax.ShapeDtypeStruct((_NC, _NROW, _H), jnp.float32),
        mesh=mesh,
        scratch_types=[
            pltpu.VMEM((_CPW, _K), jnp.int32),
            pltpu.VMEM((_CPW, _K), jnp.int32),
            pltpu.VMEM((2, _K, _H), jnp.float32),
            pltpu.VMEM_SHARED((_NROW, _H), jnp.float32),
            pltpu.SemaphoreType.DMA,
            pltpu.SemaphoreType.DMA,
        ],
    )
    return degree, aggregate


def _dinv_rows(dg_ref):
    deg = dg_ref[0, :, 0] + dg_ref[1, :, 0] + 1.0
    return lax.rsqrt(deg)[:, None]


def _tc_l1_body(dg_ref, x_ref, w_ref, out_ref):
    dinv = _dinv_rows(dg_ref)
    h = jnp.dot(x_ref[...], w_ref[...], preferred_element_type=jnp.float32)
    hs = h * dinv
    out_ref[0] = hs[:, :_H]
    out_ref[1] = hs[:, _H:]


def _tc_l2_body(dg_ref, agg_ref, b_ref, w_ref, out_ref):
    dinv = _dinv_rows(dg_ref)
    z = jnp.concatenate([agg_ref[0], agg_ref[1]], axis=1) * dinv + b_ref[...]
    h = jnp.dot(z, w_ref[...], preferred_element_type=jnp.float32) * dinv
    out_ref[0] = h[:, :_H]
    out_ref[1] = h[:, _H:]


def _tc_out_body(dg_ref, agg_ref, b_ref, out_ref):
    dinv = _dinv_rows(dg_ref)
    out_ref[...] = (jnp.concatenate([agg_ref[0], agg_ref[1]], axis=1) * dinv
                    + b_ref[...])


_GRID = (_NROW // _BN,)
_dg_spec = pl.BlockSpec((2, _BN, _DW), lambda i: (0, i, 0))
_agg_spec = pl.BlockSpec((2, _BN, _H), lambda i: (0, i, 0))
_b_spec = pl.BlockSpec((1, _D), lambda i: (0, 0))
_w_spec = pl.BlockSpec((_D, _D), lambda i: (0, 0))

_tc_l1 = pl.pallas_call(
    _tc_l1_body,
    grid=_GRID,
    in_specs=[_dg_spec, pl.BlockSpec((_BN, _D), lambda i: (i, 0)), _w_spec],
    out_specs=_agg_spec,
    out_shape=jax.ShapeDtypeStruct((2, _NROW, _H), jnp.float32),
)

_tc_l2 = pl.pallas_call(
    _tc_l2_body,
    grid=_GRID,
    in_specs=[_dg_spec, _agg_spec, _b_spec, _w_spec],
    out_specs=_agg_spec,
    out_shape=jax.ShapeDtypeStruct((2, _NROW, _H), jnp.float32),
)

_tc_out = pl.pallas_call(
    _tc_out_body,
    grid=_GRID,
    in_specs=[_dg_spec, _agg_spec, _b_spec],
    out_specs=pl.BlockSpec((_BN, _D), lambda i: (i, 0)),
    out_shape=jax.ShapeDtypeStruct((_N, _D), jnp.float32),
)


def kernel(x, edge_index, W1, b1, W2, b2):
    src = edge_index[0].astype(jnp.int32)
    dst = edge_index[1].astype(jnp.int32)
    npad = _EPAD - _E
    src_p = jnp.concatenate([src, jnp.zeros((npad,), jnp.int32)])
    dst_p = jnp.concatenate([dst, jnp.full((npad,), _N, jnp.int32)])
    src_t = src_p.reshape(_NT, _CPT, _K)
    dst_t = dst_p.reshape(_NT, _CPT, _K)
    src2 = jnp.stack([src_t, src_t + _NROW])
    dst0 = dst_p.reshape(_NC, _NT, _CP0, _K)
    ones = jnp.ones((_K, _H), jnp.float32)
    zeros = jnp.zeros((_NROW, _H), jnp.float32)
    b1r = b1.reshape(1, _D)
    b2r = b2.reshape(1, _D)

    sc_degree, sc_aggregate = _sc_kernels()
    degf = sc_degree(dst0, ones, zeros)
    deg2 = lax.slice(degf, (0, 0, 0), (_NC, _NROW, _DW))
    h1 = _tc_l1(deg2, x, W1)
    agg1 = sc_aggregate(h1.reshape(_NC * _NROW, _H), src2, dst_t)
    h2 = _tc_l2(deg2, agg1, b1r, W2)
    agg2 = sc_aggregate(h2.reshape(_NC * _NROW, _H), src2, dst_t)
    return _tc_out(deg2, agg2, b2r)


# speedup vs baseline: 1.1033x; 1.0382x over previous
import functools

import jax
import jax.numpy as jnp
from jax import lax
from jax.experimental import pallas as pl
from jax.experimental.pallas import tpu as pltpu
from jax.experimental.pallas import tpu_sc as plsc

_N = 10000
_E = 160000
_D = 256
_H = 128
_K = 128
_NT = 16
_NC = 2
_CPT = 80
_CP0 = 40
_NWIN = 2
_CPW = _CPT // _NWIN
_EPAD = _K * _NT * _CPT
_NROW = 10240
_RPT = _NROW // _NT
_BN = 640
_DW = 16


def _sc_degree_body(dst_hbm, ones_hbm, zeros_hbm, out_hbm, idx_v, ones_v, acc,
                    dsem):
    c = lax.axis_index("c")
    s = lax.axis_index("s")
    r0 = s * _RPT
    pltpu.sync_copy(zeros_hbm.at[pl.ds(r0, _RPT)], acc.at[pl.ds(r0, _RPT)])
    pltpu.sync_copy(ones_hbm, ones_v)
    pltpu.sync_copy(dst_hbm.at[c, s], idx_v)
    plsc.subcore_barrier()
    for j in range(_CP0):
        pltpu.async_copy(ones_v, acc.at[idx_v.at[j]], dsem, add=True)
    for j in range(_CP0):
        pltpu.make_async_copy(ones_v, acc.at[idx_v.at[j]], dsem).wait()
    plsc.subcore_barrier()
    pltpu.sync_copy(acc.at[pl.ds(r0, _RPT)], out_hbm.at[c, pl.ds(r0, _RPT)])


def _sc_aggregate_body(h_hbm, src_hbm, dst_hbm, out_hbm,
                       src_v, dst_v, buf, acc, sem0, sem1):
    c = lax.axis_index("c")
    s = lax.axis_index("s")
    r0 = s * _RPT
    pltpu.sync_copy(h_hbm.at[pl.ds(c * _NROW + r0, _RPT)],
                    acc.at[pl.ds(r0, _RPT)])
    plsc.subcore_barrier()

    sems = (sem0, sem1)

    def _gather(j, b):
        pltpu.async_copy(h_hbm.at[src_v.at[j]], buf.at[b], sems[b])

    def _gwait(j, b):
        pltpu.make_async_copy(h_hbm.at[src_v.at[j]], buf.at[b], sems[b]).wait()

    for win in range(_NWIN):
        pltpu.sync_copy(src_hbm.at[c, s, pl.ds(win * _CPW, _CPW)], src_v)
        pltpu.sync_copy(dst_hbm.at[s, pl.ds(win * _CPW, _CPW)], dst_v)
        _gather(0, 0)
        _gather(1, 1)
        for j in range(_CPW):
            b = j % 2
            _gwait(j, b)
            if j + 2 < _CPW:
                _gather(j + 2, b)
            pltpu.sync_copy(buf.at[b], acc.at[dst_v.at[j]], add=True)
    plsc.subcore_barrier()
    pltpu.sync_copy(acc.at[pl.ds(r0, _RPT)], out_hbm.at[c, pl.ds(r0, _RPT)])


@functools.cache
def _sc_kernels():
    mesh = plsc.VectorSubcoreMesh(core_axis_name="c", subcore_axis_name="s")
    degree = pl.kernel(
        _sc_degree_body,
        out_type=jax.ShapeDtypeStruct((_NC, _NROW, _H), jnp.float32),
        mesh=mesh,
        scratch_types=[
            pltpu.VMEM((_CP0, _K), jnp.int32),
            pltpu.VMEM((_K, _H), jnp.float32),
            pltpu.VMEM_SHARED((_NROW, _H), jnp.float32),
            pltpu.SemaphoreType.DMA,
        ],
    )
    aggregate = pl.kernel(
        _sc_aggregate_body,
        out_type=jax.ShapeDtypeStruct((_NC, _NROW, _H), jnp.float32),
        mesh=mesh,
        scratch_types=[
            pltpu.VMEM((_CPW, _K), jnp.int32),
            pltpu.VMEM((_CPW, _K), jnp.int32),
            pltpu.VMEM((2, _K, _H), jnp.float32),
            pltpu.VMEM_SHARED((_NROW, _H), jnp.float32),
            pltpu.SemaphoreType.DMA,
            pltpu.SemaphoreType.DMA,
        ],
    )
    return degree, aggregate


def _dinv_rows(dg_ref):
    deg = dg_ref[0, :, 0] + dg_ref[1, :, 0] + 1.0
    return lax.rsqrt(deg)[:, None]


def _tc_mm1_body(x_ref, w_ref, out_ref):
    h = jnp.dot(x_ref[...], w_ref[...], preferred_element_type=jnp.float32)
    out_ref[0] = h[:, :_H]
    out_ref[1] = h[:, _H:]


def _tc_scale_body(dg_ref, hr_ref, out_ref):
    dinv = _dinv_rows(dg_ref)[None]
    out_ref[...] = hr_ref[...] * dinv


def _tc_l2_body(dg_ref, agg_ref, b_ref, w_ref, out_ref):
    dinv = _dinv_rows(dg_ref)
    z = jnp.concatenate([agg_ref[0], agg_ref[1]], axis=1) * dinv + b_ref[...]
    h = jnp.dot(z, w_ref[...], preferred_element_type=jnp.float32) * dinv
    out_ref[0] = h[:, :_H]
    out_ref[1] = h[:, _H:]


def _tc_out_body(dg_ref, agg_ref, b_ref, out_ref):
    dinv = _dinv_rows(dg_ref)
    out_ref[...] = (jnp.concatenate([agg_ref[0], agg_ref[1]], axis=1) * dinv
                    + b_ref[...])


_GRID = (_NROW // _BN,)
_dg_spec = pl.BlockSpec((2, _BN, _DW), lambda i: (0, i, 0))
_agg_spec = pl.BlockSpec((2, _BN, _H), lambda i: (0, i, 0))
_b_spec = pl.BlockSpec((1, _D), lambda i: (0, 0))
_w_spec = pl.BlockSpec((_D, _D), lambda i: (0, 0))

_tc_mm1 = pl.pallas_call(
    _tc_mm1_body,
    grid=_GRID,
    in_specs=[pl.BlockSpec((_BN, _D), lambda i: (i, 0)), _w_spec],
    out_specs=_agg_spec,
    out_shape=jax.ShapeDtypeStruct((2, _NROW, _H), jnp.float32),
)

_tc_scale = pl.pallas_call(
    _tc_scale_body,
    grid=_GRID,
    in_specs=[_dg_spec, _agg_spec],
    out_specs=_agg_spec,
    out_shape=jax.ShapeDtypeStruct((2, _NROW, _H), jnp.float32),
)

_tc_l2 = pl.pallas_call(
    _tc_l2_body,
    grid=_GRID,
    in_specs=[_dg_spec, _agg_spec, _b_spec, _w_spec],
    out_specs=_agg_spec,
    out_shape=jax.ShapeDtypeStruct((2, _NROW, _H), jnp.float32),
)

_tc_out = pl.pallas_call(
    _tc_out_body,
    grid=_GRID,
    in_specs=[_dg_spec, _agg_spec, _b_spec],
    out_specs=pl.BlockSpec((_BN, _D), lambda i: (i, 0)),
    out_shape=jax.ShapeDtypeStruct((_N, _D), jnp.float32),
)


def kernel(x, edge_index, W1, b1, W2, b2):
    src = edge_index[0].astype(jnp.int32)
    dst = edge_index[1].astype(jnp.int32)
    npad = _EPAD - _E
    src_p = jnp.concatenate([src, jnp.zeros((npad,), jnp.int32)])
    dst_p = jnp.concatenate([dst, jnp.full((npad,), _N, jnp.int32)])
    src_t = src_p.reshape(_NT, _CPT, _K)
    dst_t = dst_p.reshape(_NT, _CPT, _K)
    src2 = jnp.stack([src_t, src_t + _NROW])
    dst0 = dst_p.reshape(_NC, _NT, _CP0, _K)
    ones = jnp.ones((_K, _H), jnp.float32)
    zeros = jnp.zeros((_NROW, _H), jnp.float32)
    b1r = b1.reshape(1, _D)
    b2r = b2.reshape(1, _D)

    sc_degree, sc_aggregate = _sc_kernels()
    degf = sc_degree(dst0, ones, zeros)
    deg2 = lax.slice(degf, (0, 0, 0), (_NC, _NROW, _DW))
    hr = _tc_mm1(x, W1)
    h1 = _tc_scale(deg2, hr)
    agg1 = sc_aggregate(h1.reshape(_NC * _NROW, _H), src2, dst_t)
    h2 = _tc_l2(deg2, agg1, b1r, W2)
    agg2 = sc_aggregate(h2.reshape(_NC * _NROW, _H), src2, dst_t)
    return _tc_out(deg2, agg2, b2r)
